# baseline (speedup 1.0000x reference)
import functools

import jax
import jax.numpy as jnp
from jax import lax
from jax.experimental import pallas as pl
from jax.experimental.pallas import tpu as pltpu
from jax.experimental.pallas import tpu_sc as plsc

_NUM_WORKERS = 32
_BB = 512
_LANES = 16


@functools.partial(jax.jit, static_argnames=("b", "f", "dim"))
def _gather_call(idxf, table, *, b, f, dim):
    n_total = b * f
    nbb = b // _BB
    n_chunks = f * nbb
    per_w = n_chunks // _NUM_WORKERS
    ntc = dim // 8
    ntbl = _BB // 128
    npairs = _BB * dim // _LANES
    mesh = plsc.VectorSubcoreMesh(core_axis_name="c", subcore_axis_name="s")

    @functools.partial(
        pl.kernel,
        mesh=mesh,
        out_type=jax.ShapeDtypeStruct((n_total * dim,), jnp.float32),
        scratch_types=[
            pltpu.VMEM((2, _BB), jnp.int32),
            pltpu.VMEM((2, _BB, dim), jnp.float32),
            pltpu.VMEM((2, _BB * dim), jnp.float32),
            pltpu.SemaphoreType.DMA,
            pltpu.SemaphoreType.DMA,
            pltpu.SemaphoreType.DMA,
        ],
        compiler_params=pltpu.CompilerParams(
            use_tc_tiling_on_sc=False, needs_layout_passes=False),
    )
    def emb(idx_hbm, table_hbm, out_hbm, ibuf, rows, stg, si, sg, so):
        wid = lax.axis_index("s") * 2 + lax.axis_index("c")
        iot = lax.iota(jnp.int32, _LANES)

        def chunk_id(s):
            return wid + s * _NUM_WORKERS

        def idx_off(s):
            n = chunk_id(s)
            fi, bb = n // nbb, n % nbb
            return pl.multiple_of(fi * b + bb * _BB, _BB)

        def start_idx(s, buf):
            return pltpu.async_copy(
                idx_hbm.at[pl.ds(idx_off(s), _BB)], ibuf.at[buf], si)

        def start_gather(s, buf):
            return pltpu.async_copy(
                table_hbm.at[ibuf.at[buf]], rows.at[buf], sg)

        def transpose_chunk(buf):
            rbuf = rows.at[buf]
            sbuf = stg.at[buf]

            @plsc.parallel_loop(0, npairs, unroll=2)
            def _(k):
                tc = k // (npairs // ntc)
                rem = k % (npairs // ntc)
                tbl = rem // (npairs // (ntc * ntbl))
                ci = (rem // 8) % 8
                bjg = k % 8
                i_row = tbl * 128 + bjg * _LANES + iot
                i_col = jnp.zeros((_LANES,), jnp.int32) + (tc * 8 + ci)
                vec = plsc.load_gather(rbuf, [i_row, i_col])
                pos = tc * (_BB * 8) + tbl * 1024 + ci * 128 + bjg * _LANES
                sbuf[pl.ds(pos, _LANES)] = vec

        def start_out(s, buf):
            n = chunk_id(s)
            fi, bb = n // nbb, n % nbb
            handles = []
            for tc in range(ntc):
                off = pl.multiple_of(
                    (fi * ntc + tc) * (b * 8) + bb * (_BB * 8), _BB * 8)
                handles.append(pltpu.async_copy(
                    stg.at[buf].at[pl.ds(tc * (_BB * 8), _BB * 8)],
                    out_hbm.at[pl.ds(off, _BB * 8)], so))
            return handles

        idx_cp = [None] * per_w
        g_cp = [None] * per_w
        o_cp = [None] * per_w
        pltpu.sync_copy(idx_hbm.at[pl.ds(idx_off(0), _BB)], ibuf.at[0])
        g_cp[0] = start_gather(0, 0)
        if per_w > 1:
            idx_cp[1] = start_idx(1, 1)
        for s in range(per_w):
            cur = s % 2
            if s + 1 < per_w:
                idx_cp[s + 1].wait()
                g_cp[s + 1] = start_gather(s + 1, cur ^ 1)
            g_cp[s].wait()
            if s + 2 < per_w:
                idx_cp[s + 2] = start_idx(s + 2, cur)
            if s >= 2:
                for h in o_cp[s - 2]:
                    h.wait()
            transpose_chunk(cur)
            o_cp[s] = start_out(s, cur)
        for s in (per_w - 2, per_w - 1):
            if s >= 0:
                for h in o_cp[s]:
                    h.wait()

    return emb(idxf, table)


def kernel(input, weight):
    b, f = input.shape
    v, dim = weight.shape
    idxf = input.T.reshape(b * f).astype(jnp.int32)
    out1d = _gather_call(idxf, weight, b=b, f=f, dim=dim)
    t = out1d.reshape(f, dim // 8, b // 128, 8, 128)
    t = t.transpose(2, 4, 0, 1, 3)
    return t.reshape(b, f, dim)


# speedup vs baseline: 1.1210x; 1.1210x over previous
import functools

import jax
import jax.numpy as jnp
from jax import lax
from jax.experimental import pallas as pl
from jax.experimental.pallas import tpu as pltpu
from jax.experimental.pallas import tpu_sc as plsc

_NUM_WORKERS = 32
_BB = 512
_LANES = 16


@functools.partial(jax.jit, static_argnames=("b", "f", "dim"))
def _gather_call(idxf, table, *, b, f, dim):
    n_total = b * f
    nbb = b // _BB
    n_chunks = f * nbb
    per_w = n_chunks // _NUM_WORKERS
    ntc = dim // 8
    ntbl = _BB // 128
    npairs = _BB * dim // _LANES
    mesh = plsc.VectorSubcoreMesh(core_axis_name="c", subcore_axis_name="s")

    @functools.partial(
        pl.kernel,
        mesh=mesh,
        out_type=jax.ShapeDtypeStruct((n_total * dim,), jnp.float32),
        scratch_types=[
            pltpu.VMEM((2, _BB), jnp.int32),
            pltpu.VMEM((2, _BB, dim), jnp.float32),
            pltpu.VMEM((2, _BB * dim), jnp.float32),
            pltpu.SemaphoreType.DMA,
            pltpu.SemaphoreType.DMA,
            pltpu.SemaphoreType.DMA,
        ],
        compiler_params=pltpu.CompilerParams(
            use_tc_tiling_on_sc=False, needs_layout_passes=False),
    )
    def emb(idx_hbm, table_hbm, out_hbm, ibuf, rows, stg, si, sg, so):
        wid = lax.axis_index("s") * 2 + lax.axis_index("c")
        iot = lax.iota(jnp.int32, _LANES)

        def chunk_id(s):
            return wid + s * _NUM_WORKERS

        def idx_off(s):
            n = chunk_id(s)
            fi, bb = n // nbb, n % nbb
            return pl.multiple_of(fi * b + bb * _BB, _BB)

        def start_idx(s, buf):
            return pltpu.async_copy(
                idx_hbm.at[pl.ds(idx_off(s), _BB)], ibuf.at[buf], si)

        def start_gather(s, buf):
            return pltpu.async_copy(
                table_hbm.at[ibuf.at[buf]], rows.at[buf], sg)

        base0 = (iot // 8) * (_BB * 8) + (iot % 8) * 128
        base1 = ((iot + _LANES) // 8) * (_BB * 8) + ((iot + _LANES) % 8) * 128

        def transpose_chunk(buf):
            rbuf = rows.at[buf]
            sbuf = stg.at[buf]

            @plsc.parallel_loop(0, _BB, unroll=4)
            def _(j):
                off = (j // 128) * 1024 + (j % 128)
                v0 = rbuf[j, pl.ds(0, _LANES)]
                v1 = rbuf[j, pl.ds(_LANES, _LANES)]
                plsc.store_scatter(sbuf, [base0 + off], v0)
                plsc.store_scatter(sbuf, [base1 + off], v1)

        def start_out(s, buf):
            n = chunk_id(s)
            fi, bb = n // nbb, n % nbb
            handles = []
            for tc in range(ntc):
                off = pl.multiple_of(
                    (fi * ntc + tc) * (b * 8) + bb * (_BB * 8), _BB * 8)
                handles.append(pltpu.async_copy(
                    stg.at[buf].at[pl.ds(tc * (_BB * 8), _BB * 8)],
                    out_hbm.at[pl.ds(off, _BB * 8)], so))
            return handles

        idx_cp = [None] * per_w
        g_cp = [None] * per_w
        o_cp = [None] * per_w
        pltpu.sync_copy(idx_hbm.at[pl.ds(idx_off(0), _BB)], ibuf.at[0])
        g_cp[0] = start_gather(0, 0)
        if per_w > 1:
            idx_cp[1] = start_idx(1, 1)
        for s in range(per_w):
            cur = s % 2
            if s + 1 < per_w:
                idx_cp[s + 1].wait()
                g_cp[s + 1] = start_gather(s + 1, cur ^ 1)
            g_cp[s].wait()
            if s + 2 < per_w:
                idx_cp[s + 2] = start_idx(s + 2, cur)
            if s >= 2:
                for h in o_cp[s - 2]:
                    h.wait()
            transpose_chunk(cur)
            o_cp[s] = start_out(s, cur)
        for s in (per_w - 2, per_w - 1):
            if s >= 0:
                for h in o_cp[s]:
                    h.wait()

    return emb(idxf, table)


def kernel(input, weight):
    b, f = input.shape
    v, dim = weight.shape
    idxf = input.T.reshape(b * f).astype(jnp.int32)
    out1d = _gather_call(idxf, weight, b=b, f=f, dim=dim)
    t = out1d.reshape(f, dim // 8, b // 128, 8, 128)
    t = t.transpose(2, 4, 0, 1, 3)
    return t.reshape(b, f, dim)
